# Initial kernel scaffold; baseline (speedup 1.0000x reference)
import jax
import jax.numpy as jnp
from jax.experimental import pallas as pl


def kernel(article_id, prod_name_tokens, article_table, text_table):
    raise NotImplementedError("write your pallas kernel here")


# speedup vs baseline: 12.5043x; 12.5043x over previous
import functools

import jax
import jax.numpy as jnp
from jax import lax
from jax.experimental import pallas as pl
from jax.experimental.pallas import tpu as pltpu
from jax.experimental.pallas import tpu_sc as plsc

B = 16384
L = 20
EMBED = 32
TEXT_VOCAB = 10000

NC, NS = 2, 16
NW = NC * NS
ROWS_W = B // NW
CHUNK = 32
NCHUNK = ROWS_W // CHUNK
TOK_W = ROWS_W * L
IDXC = 128
TOK_IDX_ROWS = TOK_W // IDXC
ART_IDX_ROWS = ROWS_W // IDXC
BURSTS = CHUNK * L // IDXC
ZROW = TEXT_VOCAB

_mesh = plsc.VectorSubcoreMesh(core_axis_name="c", subcore_axis_name="s")


@functools.partial(
    pl.kernel,
    out_type=jax.ShapeDtypeStruct((B, 2 * EMBED), jnp.float32),
    mesh=_mesh,
    compiler_params=pltpu.CompilerParams(
        needs_layout_passes=False, use_tc_tiling_on_sc=False),
    scratch_types=[
        pltpu.VMEM((TOK_IDX_ROWS, IDXC), jnp.int32),
        pltpu.VMEM((ART_IDX_ROWS, IDXC), jnp.int32),
        pltpu.VMEM((ROWS_W, EMBED), jnp.float32),
        pltpu.VMEM((CHUNK * L, EMBED), jnp.float32),
        pltpu.VMEM((ROWS_W,), jnp.float32),
        pltpu.VMEM((CHUNK, 2 * EMBED), jnp.float32),
        pltpu.SemaphoreType.DMA,
        pltpu.SemaphoreType.DMA,
    ],
)
def _article_kernel(tok_hbm, artid_hbm, art_tab, txt_tab, out_hbm,
                    tokidx, artidx, artrows, tokrows, rcnt, outbuf,
                    sem_tok, sem_art):
    wid = lax.axis_index("s") * NC + lax.axis_index("c")
    base = wid * ROWS_W

    pltpu.sync_copy(tok_hbm.at[pl.ds(wid * TOK_IDX_ROWS, TOK_IDX_ROWS), :],
                    tokidx)
    pltpu.sync_copy(artid_hbm.at[pl.ds(wid * ART_IDX_ROWS, ART_IDX_ROWS), :],
                    artidx)

    art_copies = []
    for j in range(ART_IDX_ROWS):
        art_copies.append(pltpu.async_copy(
            art_tab.at[artidx.at[j]],
            artrows.at[pl.ds(j * IDXC, IDXC), :],
            sem_art))

    lane = lax.iota(jnp.int32, 16)

    def cnt_body(g, carry):
        p0 = (g * 16 + lane) * L
        cntf = jnp.zeros((16,), jnp.float32)
        for l in range(L):
            p = p0 + l
            r = jnp.right_shift(p, 7)
            col = jnp.bitwise_and(p, IDXC - 1)
            t = plsc.load_gather(tokidx, [r, col])
            nz = t != 0
            cntf = cntf + nz.astype(jnp.float32)
            plsc.store_scatter(tokidx, [r, col],
                               jnp.where(nz, t, jnp.int32(ZROW)))
        rcnt[pl.ds(g * 16, 16)] = 1.0 / jnp.maximum(cntf, 1.0)
        return carry

    lax.fori_loop(0, ROWS_W // 16, cnt_body, 0)

    for cp in art_copies:
        cp.wait()

    def chunk_body(c, carry):
        copies = []
        for j in range(BURSTS):
            copies.append(pltpu.async_copy(
                txt_tab.at[tokidx.at[c * BURSTS + j]],
                tokrows.at[pl.ds(j * IDXC, IDXC), :],
                sem_tok))
        for cp in copies:
            cp.wait()

        def row_body(r, rcarry):
            g = c * CHUNK + r
            acc0 = tokrows[r * L, pl.ds(0, 16)]
            acc1 = tokrows[r * L, pl.ds(16, 16)]
            for l in range(1, L):
                acc0 = acc0 + tokrows[r * L + l, pl.ds(0, 16)]
                acc1 = acc1 + tokrows[r * L + l, pl.ds(16, 16)]
            bc = plsc.load_gather(rcnt, [jnp.zeros((16,), jnp.int32) + g])
            outbuf[r, pl.ds(0, 16)] = artrows[g, pl.ds(0, 16)]
            outbuf[r, pl.ds(16, 16)] = artrows[g, pl.ds(16, 16)]
            outbuf[r, pl.ds(32, 16)] = acc0 * bc
            outbuf[r, pl.ds(48, 16)] = acc1 * bc
            return rcarry

        lax.fori_loop(0, CHUNK, row_body, 0)
        pltpu.sync_copy(outbuf, out_hbm.at[pl.ds(base + c * CHUNK, CHUNK), :])
        return carry

    lax.fori_loop(0, NCHUNK, chunk_body, 0)


def kernel(article_id, prod_name_tokens, article_table, text_table):
    txt_aug = jnp.concatenate(
        [text_table, jnp.zeros((8, EMBED), text_table.dtype)], axis=0)
    tok2d = prod_name_tokens.reshape(-1, IDXC)
    art2d = article_id.reshape(-1, IDXC)
    return _article_kernel(tok2d, art2d, article_table, txt_aug)


# speedup vs baseline: 14.8714x; 1.1893x over previous
import functools

import jax
import jax.numpy as jnp
from jax import lax
from jax.experimental import pallas as pl
from jax.experimental.pallas import tpu as pltpu
from jax.experimental.pallas import tpu_sc as plsc

B = 16384
L = 20
EMBED = 32

NC, NS = 2, 16
NW = NC * NS
ROWS_W = B // NW
CHUNK = 32
NCHUNK = ROWS_W // CHUNK
TOK_W = ROWS_W * L
IDXC = 128
TOK_IDX_ROWS = TOK_W // IDXC
ART_IDX_ROWS = ROWS_W // IDXC
BURSTS = CHUNK * L // IDXC

_mesh = plsc.VectorSubcoreMesh(core_axis_name="c", subcore_axis_name="s")


@functools.partial(
    pl.kernel,
    out_type=jax.ShapeDtypeStruct((B, 2 * EMBED), jnp.float32),
    mesh=_mesh,
    compiler_params=pltpu.CompilerParams(
        needs_layout_passes=False, use_tc_tiling_on_sc=False),
    scratch_types=[
        pltpu.VMEM((TOK_IDX_ROWS, IDXC), jnp.int32),
        pltpu.VMEM((ART_IDX_ROWS, IDXC), jnp.int32),
        pltpu.VMEM((ROWS_W, EMBED), jnp.float32),
        pltpu.VMEM((2, CHUNK * L, EMBED), jnp.float32),
        pltpu.VMEM((ROWS_W,), jnp.float32),
        pltpu.VMEM((ROWS_W,), jnp.float32),
        pltpu.VMEM((1, EMBED), jnp.float32),
        pltpu.VMEM((2, CHUNK, 2 * EMBED), jnp.float32),
        pltpu.SemaphoreType.DMA,
        pltpu.SemaphoreType.DMA,
        pltpu.SemaphoreType.DMA,
        pltpu.SemaphoreType.DMA,
    ],
)
def _article_kernel(tok_hbm, artid_hbm, art_tab, txt_tab, out_hbm,
                    tokidx, artidx, artrows, tokrows, rcnt, zcnt, row0,
                    outbuf, sem_a, sem_b, sem_art, sem_out):
    wid = lax.axis_index("s") * NC + lax.axis_index("c")
    base = wid * ROWS_W

    pltpu.sync_copy(tok_hbm.at[pl.ds(wid * TOK_IDX_ROWS, TOK_IDX_ROWS), :],
                    tokidx)
    pltpu.sync_copy(artid_hbm.at[pl.ds(wid * ART_IDX_ROWS, ART_IDX_ROWS), :],
                    artidx)
    pltpu.sync_copy(txt_tab.at[pl.ds(0, 1), :], row0)

    art_copies = []
    for j in range(ART_IDX_ROWS):
        art_copies.append(pltpu.async_copy(
            art_tab.at[artidx.at[j]],
            artrows.at[pl.ds(j * IDXC, IDXC), :],
            sem_art))

    lane = lax.iota(jnp.int32, 16)

    def cnt_body(g, carry):
        p0 = (g * 16 + lane) * L
        cntf = jnp.zeros((16,), jnp.float32)
        for l in range(L):
            p = p0 + l
            r = jnp.right_shift(p, 7)
            col = jnp.bitwise_and(p, IDXC - 1)
            t = plsc.load_gather(tokidx, [r, col])
            cntf = cntf + (t != 0).astype(jnp.float32)
        rcnt[pl.ds(g * 16, 16)] = 1.0 / jnp.maximum(cntf, 1.0)
        zcnt[pl.ds(g * 16, 16)] = jnp.float32(L) - cntf
        return carry

    lax.fori_loop(0, ROWS_W // 16, cnt_body, 0)

    r0a = row0[0, pl.ds(0, 16)]
    r0b = row0[0, pl.ds(16, 16)]

    for cp in art_copies:
        cp.wait()

    def fire_chunk(c, sem):
        par = c % 2
        return [pltpu.async_copy(
            txt_tab.at[tokidx.at[c * BURSTS + j]],
            tokrows.at[par, pl.ds(j * IDXC, IDXC), :],
            sem) for j in range(BURSTS)]

    tok_sems = (sem_a, sem_b)
    pending = fire_chunk(0, tok_sems[0])
    out_copies = [None, None]

    for c in range(NCHUNK):
        par = c % 2
        if c + 1 < NCHUNK:
            next_copies = fire_chunk(c + 1, tok_sems[(c + 1) % 2])
        for cp in pending:
            cp.wait()
        if c + 1 < NCHUNK:
            pending = next_copies
        if out_copies[par] is not None:
            out_copies[par].wait()

        def row_body(r, rcarry, _par=par, _c=c):
            g = _c * CHUNK + r
            acc0 = tokrows[_par, r * L, pl.ds(0, 16)]
            acc1 = tokrows[_par, r * L, pl.ds(16, 16)]
            for l in range(1, L):
                acc0 = acc0 + tokrows[_par, r * L + l, pl.ds(0, 16)]
                acc1 = acc1 + tokrows[_par, r * L + l, pl.ds(16, 16)]
            gidx = jnp.zeros((16,), jnp.int32) + g
            rc = plsc.load_gather(rcnt, [gidx])
            zc = plsc.load_gather(zcnt, [gidx])
            outbuf[_par, r, pl.ds(0, 16)] = artrows[g, pl.ds(0, 16)]
            outbuf[_par, r, pl.ds(16, 16)] = artrows[g, pl.ds(16, 16)]
            outbuf[_par, r, pl.ds(32, 16)] = (acc0 - zc * r0a) * rc
            outbuf[_par, r, pl.ds(48, 16)] = (acc1 - zc * r0b) * rc
            return rcarry

        lax.fori_loop(0, CHUNK, row_body, 0)
        out_copies[par] = pltpu.async_copy(
            outbuf.at[par],
            out_hbm.at[pl.ds(base + c * CHUNK, CHUNK), :],
            sem_out)

    for cp in out_copies:
        if cp is not None:
            cp.wait()


def kernel(article_id, prod_name_tokens, article_table, text_table):
    tok2d = prod_name_tokens.reshape(-1, IDXC)
    art2d = article_id.reshape(-1, IDXC)
    return _article_kernel(tok2d, art2d, article_table, text_table)
